# baseline (speedup 1.0000x reference)
import jax
import jax.numpy as jnp
from jax import lax
from jax.experimental import pallas as pl
from jax.experimental.pallas import tpu as pltpu
from jax.experimental.pallas import tpu_sc as plsc

_USER_SIZE = 100000
_REL_SIZE = 100
_MBL_REL_SIZE = 7
_DIM = 32
_KGE_MARGIN = 1.0
_NC, _NS, _L = 2, 16, 16
_NW = _NC * _NS
_CHUNK = 64
_LINE = 128


def _fsqrt(x):
    i = plsc.bitcast(x, jnp.int32)
    y = plsc.bitcast(jnp.int32(0x5F3759DF) - (i >> 1), jnp.float32)
    for _ in range(3):
        y = y * (1.5 - 0.5 * x * y * y)
    return jnp.where(x > 0.0, x * y, 0.0)


def _make_body(n_chunks):
    bpw = n_chunks * _CHUNK

    def body(u_hbm, i_hbm, j_hbm, inter_hbm, kgp_hbm, kgn_hbm, um1_hbm,
             im1_hbm, mbl_hbm, relmb_hbm, epl_hbm, relep_hbm, out_hbm,
             u_ix, i_ix, j_ix,
             u_sx, mbi_sx, mbj_sx, kph_sx, kpt_sx, knh_sx, knt_sx,
             epu_sx, epi_sx, epj_sx,
             u_of, mbi_of, mbj_of, kph_of, kpt_of, knh_of, knt_of,
             epu_of, epi_of, epj_of,
             mbb, rp, rn,
             kgp_v, kgn_v, inter_v, relmb_v, relep_v, rowbuf, loss_v,
             sem_a, sem_b):
        wid = lax.axis_index("s") * _NC + lax.axis_index("c")
        base = pl.multiple_of(wid * bpw, _CHUNK)
        iota = lax.iota(jnp.int32, _L)

        hs = []
        for c in range(n_chunks):
            off = pl.multiple_of(base + c * _CHUNK, _CHUNK)
            hs.append(pltpu.async_copy(u_hbm.at[pl.ds(off, _CHUNK)],
                                       u_ix.at[c], sem_a))
            hs.append(pltpu.async_copy(i_hbm.at[pl.ds(off, _CHUNK)],
                                       i_ix.at[c], sem_a))
            hs.append(pltpu.async_copy(j_hbm.at[pl.ds(off, _CHUNK)],
                                       j_ix.at[c], sem_a))
        base3 = pl.multiple_of(wid * bpw * 3, 8)
        hs.append(pltpu.async_copy(kgp_hbm.at[pl.ds(base3, bpw * 3)],
                                   kgp_v, sem_a))
        hs.append(pltpu.async_copy(kgn_hbm.at[pl.ds(base3, bpw * 3)],
                                   kgn_v, sem_a))
        hs.append(pltpu.async_copy(inter_hbm.at[pl.ds(base3, bpw * 3)],
                                   inter_v, sem_a))
        hs.append(pltpu.async_copy(relmb_hbm, relmb_v, sem_a))
        hs.append(pltpu.async_copy(relep_hbm, relep_v, sem_a))
        for h in hs:
            h.wait()

        ng = _CHUNK // _L

        def split(v):
            return v >> 2, (v & 3) << 5

        def prep(c):
            def step(g, carry):
                off = g * _L
                li3 = (c * _CHUNK + off + iota) * 3
                u_raw = u_ix[c, pl.ds(off, _L)]
                i_raw = i_ix[c, pl.ds(off, _L)]
                j_raw = j_ix[c, pl.ds(off, _L)]
                s, o = split(u_raw)
                u_sx[c, pl.ds(off, _L)] = s
                u_of[c, pl.ds(off, _L)] = o
                s, o = split(i_raw + _USER_SIZE)
                mbi_sx[c, pl.ds(off, _L)] = s
                mbi_of[c, pl.ds(off, _L)] = o
                s, o = split(j_raw + _USER_SIZE)
                mbj_sx[c, pl.ds(off, _L)] = s
                mbj_of[c, pl.ds(off, _L)] = o
                s, o = split(plsc.load_gather(kgp_v, [li3]))
                kph_sx[c, pl.ds(off, _L)] = s
                kph_of[c, pl.ds(off, _L)] = o
                s, o = split(plsc.load_gather(kgp_v, [li3 + 2]))
                kpt_sx[c, pl.ds(off, _L)] = s
                kpt_of[c, pl.ds(off, _L)] = o
                s, o = split(plsc.load_gather(kgn_v, [li3]))
                knh_sx[c, pl.ds(off, _L)] = s
                knh_of[c, pl.ds(off, _L)] = o
                s, o = split(plsc.load_gather(kgn_v, [li3 + 2]))
                knt_sx[c, pl.ds(off, _L)] = s
                knt_of[c, pl.ds(off, _L)] = o
                rp[pl.ds(c * _CHUNK + off, _L)] = (
                    plsc.load_gather(kgp_v, [li3 + 1]) % _REL_SIZE)
                rn[pl.ds(c * _CHUNK + off, _L)] = (
                    plsc.load_gather(kgn_v, [li3 + 1]) % _REL_SIZE)
                b0 = plsc.load_gather(inter_v, [li3])
                b1 = plsc.load_gather(inter_v, [li3 + 1])
                b2 = plsc.load_gather(inter_v, [li3 + 2])
                bb = b0 + 2 * b1 + 4 * b2 - 1
                mbb[pl.ds(c * _CHUNK + off, _L)] = jnp.minimum(
                    jnp.maximum(bb, 0), _MBL_REL_SIZE - 1)
                return carry
            lax.fori_loop(0, ng, step, 0)

        for c in range(n_chunks):
            prep(c)

        hs = []
        for c in range(n_chunks):
            hs.append(pltpu.async_copy(um1_hbm.at[u_ix.at[c]],
                                       epu_sx.at[c], sem_a))
            hs.append(pltpu.async_copy(im1_hbm.at[i_ix.at[c]],
                                       epi_sx.at[c], sem_a))
            hs.append(pltpu.async_copy(im1_hbm.at[j_ix.at[c]],
                                       epj_sx.at[c], sem_a))
        for h in hs:
            h.wait()

        def prep2(c):
            def step(g, carry):
                off = g * _L
                for sx, of in ((epu_sx, epu_of), (epi_sx, epi_of),
                               (epj_sx, epj_of)):
                    raw = sx[c, pl.ds(off, _L)]
                    s, o = split(raw)
                    sx[c, pl.ds(off, _L)] = s
                    of[c, pl.ds(off, _L)] = o
                return carry
            lax.fori_loop(0, ng, step, 0)

        for c in range(n_chunks):
            prep2(c)

        srcs_of = (u_of, mbi_of, mbj_of, kph_of, kpt_of, knh_of, knt_of,
                   epu_of, epi_of, epj_of)

        def fire(c):
            srcs = (
                mbl_hbm.at[u_sx.at[c]], mbl_hbm.at[mbi_sx.at[c]],
                mbl_hbm.at[mbj_sx.at[c]],
                epl_hbm.at[kph_sx.at[c]], epl_hbm.at[kpt_sx.at[c]],
                epl_hbm.at[knh_sx.at[c]], epl_hbm.at[knt_sx.at[c]],
                epl_hbm.at[epu_sx.at[c]], epl_hbm.at[epi_sx.at[c]],
                epl_hbm.at[epj_sx.at[c]],
            )
            return [pltpu.async_copy(s, rowbuf.at[r], sem_b)
                    for r, s in enumerate(srcs)]

        def compute_chunk(c, loss_acc):
            def group(g, acc):
                off = g * _L
                rows16 = off + iota
                bvec = mbb[pl.ds(c * _CHUNK + off, _L)]
                rpv = rp[pl.ds(c * _CHUNK + off, _L)]
                rnv = rn[pl.ds(c * _CHUNK + off, _L)]
                lanes = [of[c, pl.ds(off, _L)] for of in srcs_of]
                zeros = jnp.zeros((_L,), jnp.float32)

                def kstep(k, accs):
                    a1, a2, a3, a4, a5, a6, a7 = accs
                    ck = jnp.full((_L,), k, jnp.int32)

                    def row(r):
                        return plsc.load_gather(
                            rowbuf,
                            [jnp.full((_L,), r, jnp.int32), rows16,
                             lanes[r] + k])

                    vhu, vti, vtj = row(0), row(1), row(2)
                    vr = plsc.load_gather(relmb_v, [bvec, ck])
                    vhp, vtp = row(3), row(4)
                    vhn, vtn = row(5), row(6)
                    vrp = plsc.load_gather(relep_v, [rpv, ck])
                    vrn = plsc.load_gather(relep_v, [rnv, ck])
                    veu, vei, vej = row(7), row(8), row(9)
                    s = vhu + vr
                    d1 = s - vti
                    d2 = s - vtj
                    e1 = vhp + vrp - vtp
                    e2 = vhn + vrn - vtn
                    vu = vhu + veu
                    f1 = vu - (vti + vei)
                    f2 = vu - (vtj + vej)
                    return (a1 + d1 * d1, a2 + d2 * d2, a3 + e1 * e1,
                            a4 + e2 * e2, a5 + f1 * f1, a6 + f2 * f2,
                            a7 + vr * vr)

                a1, a2, a3, a4, a5, a6, a7 = lax.fori_loop(
                    0, _DIM, kstep, (zeros,) * 7, unroll=2)
                pd, nd = _fsqrt(a1), _fsqrt(a2)
                pe, ne = _fsqrt(a3), _fsqrt(a4)
                dp, dn = _fsqrt(a5), _fsqrt(a6)
                rno = _fsqrt(a7)
                m = 1.5 - 1.0 / (1.0 + jnp.exp(-rno))
                t = (jnp.maximum(pd - nd + _KGE_MARGIN, 0.0)
                     + jnp.maximum(pe - ne + _KGE_MARGIN, 0.0)
                     + jnp.maximum(dp - dn + m, 0.0))
                return acc + t

            return lax.fori_loop(0, ng, group, loss_acc)

        loss_acc = jnp.zeros((_L,), jnp.float32)
        for c in range(n_chunks):
            hs = fire(c)
            for h in hs:
                h.wait()
            loss_acc = compute_chunk(c, loss_acc)

        loss_v[...] = loss_acc
        pltpu.sync_copy(loss_v, out_hbm.at[wid])

    return body


def kernel(u_batch, i_batch, j_batch, interactions, kg_pos, kg_neg,
           user_entity_map, item_entity_map,
           mbl_entities, mbl_relations, epl_entities, epl_relations):
    batch = u_batch.shape[0]
    n_chunks = batch // (_NW * _CHUNK)
    bpw = n_chunks * _CHUNK

    u32 = u_batch.astype(jnp.int32)
    i32 = i_batch.astype(jnp.int32)
    j32 = j_batch.astype(jnp.int32)
    inter_flat = interactions.astype(jnp.int32).reshape(-1)
    kgp_flat = kg_pos.astype(jnp.int32).reshape(-1)
    kgn_flat = kg_neg.astype(jnp.int32).reshape(-1)
    um1 = user_entity_map[:, 1].astype(jnp.int32)
    im1 = item_entity_map[:, 1].astype(jnp.int32)
    mbl_lines = jnp.concatenate(
        [mbl_entities[0::4], mbl_entities[1::4],
         mbl_entities[2::4], mbl_entities[3::4]], axis=1)
    epl_lines = jnp.concatenate(
        [epl_entities[0::4], epl_entities[1::4],
         epl_entities[2::4], epl_entities[3::4]], axis=1)

    scratch = (
        [pltpu.VMEM((n_chunks, _CHUNK), jnp.int32)] * 23
        + [pltpu.VMEM((bpw,), jnp.int32)] * 3
        + [pltpu.VMEM((bpw * 3,), jnp.int32)] * 3
        + [pltpu.VMEM((_MBL_REL_SIZE, _DIM), jnp.float32),
           pltpu.VMEM((_REL_SIZE, _DIM), jnp.float32),
           pltpu.VMEM((10, _CHUNK, _LINE), jnp.float32),
           pltpu.VMEM((_L,), jnp.float32),
           pltpu.SemaphoreType.DMA, pltpu.SemaphoreType.DMA]
    )
    out = pl.kernel(
        _make_body(n_chunks),
        out_type=jax.ShapeDtypeStruct((_NW, _L), jnp.float32),
        mesh=plsc.VectorSubcoreMesh(core_axis_name="c", subcore_axis_name="s",
                                    num_cores=_NC, num_subcores=_NS),
        compiler_params=pltpu.CompilerParams(needs_layout_passes=False,
                                             use_tc_tiling_on_sc=True),
        scratch_types=scratch,
    )(u32, i32, j32, inter_flat, kgp_flat, kgn_flat, um1, im1,
      mbl_lines, mbl_relations, epl_lines, epl_relations)
    return jnp.sum(out) * (1.0 / batch)


# speedup vs baseline: 8.0431x; 8.0431x over previous
import jax
import jax.numpy as jnp
from jax import lax
from jax.experimental import pallas as pl
from jax.experimental.pallas import tpu as pltpu
from jax.experimental.pallas import tpu_sc as plsc

_USER_SIZE = 100000
_REL_SIZE = 100
_MBL_REL_SIZE = 7
_DIM = 32
_KGE_MARGIN = 1.0
_NC, _NS, _L = 2, 16, 16
_NW = _NC * _NS
_CHUNK = 128


def _fsqrt(x):
    i = plsc.bitcast(x, jnp.int32)
    y = plsc.bitcast(jnp.int32(0x5F3759DF) - (i >> 1), jnp.float32)
    for _ in range(3):
        y = y * (1.5 - 0.5 * x * y * y)
    return jnp.where(x > 0.0, x * y, 0.0)


def _make_body(n_chunks):
    bpw = n_chunks * _CHUNK

    def body(u_hbm, i_hbm, j_hbm, inter_hbm, kgp_hbm, kgn_hbm, um1_hbm,
             im1_hbm, mbl_hbm, relmb_hbm, epl_hbm, relep_hbm, out_hbm,
             u_ix, i_ix, j_ix, mbi_ix, mbj_ix, kph_ix, kpt_ix, knh_ix,
             knt_ix, epu_ix, epi_ix, epj_ix, mbb, rp, rn,
             kgp_v, kgn_v, inter_v, relmb_v, relep_v, rowbuf, loss_v,
             sem_a, sem_b):
        wid = lax.axis_index("s") * _NC + lax.axis_index("c")
        base = pl.multiple_of(wid * bpw, _CHUNK)
        iota = lax.iota(jnp.int32, _L)

        hs = []
        for c in range(n_chunks):
            off = pl.multiple_of(base + c * _CHUNK, _CHUNK)
            hs.append(pltpu.async_copy(u_hbm.at[pl.ds(off, _CHUNK)],
                                       u_ix.at[c], sem_a))
            hs.append(pltpu.async_copy(i_hbm.at[pl.ds(off, _CHUNK)],
                                       i_ix.at[c], sem_a))
            hs.append(pltpu.async_copy(j_hbm.at[pl.ds(off, _CHUNK)],
                                       j_ix.at[c], sem_a))
        base3 = pl.multiple_of(wid * bpw * 3, 8)
        hs.append(pltpu.async_copy(kgp_hbm.at[pl.ds(base3, bpw * 3)],
                                   kgp_v, sem_a))
        hs.append(pltpu.async_copy(kgn_hbm.at[pl.ds(base3, bpw * 3)],
                                   kgn_v, sem_a))
        hs.append(pltpu.async_copy(inter_hbm.at[pl.ds(base3, bpw * 3)],
                                   inter_v, sem_a))
        hs.append(pltpu.async_copy(relmb_hbm, relmb_v, sem_a))
        hs.append(pltpu.async_copy(relep_hbm, relep_v, sem_a))
        for h in hs:
            h.wait()

        ng = _CHUNK // _L

        def prep(c):
            def step(g, carry):
                off = g * _L
                li3 = (c * _CHUNK + off + iota) * 3
                i_raw = i_ix[c, pl.ds(off, _L)]
                j_raw = j_ix[c, pl.ds(off, _L)]
                mbi_ix[c, pl.ds(off, _L)] = i_raw + _USER_SIZE
                mbj_ix[c, pl.ds(off, _L)] = j_raw + _USER_SIZE
                kph_ix[c, pl.ds(off, _L)] = plsc.load_gather(kgp_v, [li3])
                kpt_ix[c, pl.ds(off, _L)] = plsc.load_gather(kgp_v, [li3 + 2])
                knh_ix[c, pl.ds(off, _L)] = plsc.load_gather(kgn_v, [li3])
                knt_ix[c, pl.ds(off, _L)] = plsc.load_gather(kgn_v, [li3 + 2])
                rp[pl.ds(c * _CHUNK + off, _L)] = (
                    plsc.load_gather(kgp_v, [li3 + 1]) % _REL_SIZE)
                rn[pl.ds(c * _CHUNK + off, _L)] = (
                    plsc.load_gather(kgn_v, [li3 + 1]) % _REL_SIZE)
                b0 = plsc.load_gather(inter_v, [li3])
                b1 = plsc.load_gather(inter_v, [li3 + 1])
                b2 = plsc.load_gather(inter_v, [li3 + 2])
                bb = b0 + 2 * b1 + 4 * b2 - 1
                mbb[pl.ds(c * _CHUNK + off, _L)] = jnp.minimum(
                    jnp.maximum(bb, 0), _MBL_REL_SIZE - 1)
                return carry
            lax.fori_loop(0, ng, step, 0)

        for c in range(n_chunks):
            prep(c)

        hs = []
        for c in range(n_chunks):
            hs.append(pltpu.async_copy(um1_hbm.at[u_ix.at[c]],
                                       epu_ix.at[c], sem_a))
            hs.append(pltpu.async_copy(im1_hbm.at[i_ix.at[c]],
                                       epi_ix.at[c], sem_a))
            hs.append(pltpu.async_copy(im1_hbm.at[j_ix.at[c]],
                                       epj_ix.at[c], sem_a))
        for h in hs:
            h.wait()

        def fire(c):
            srcs = (
                mbl_hbm.at[u_ix.at[c]], mbl_hbm.at[mbi_ix.at[c]],
                mbl_hbm.at[mbj_ix.at[c]],
                epl_hbm.at[kph_ix.at[c]], epl_hbm.at[kpt_ix.at[c]],
                epl_hbm.at[knh_ix.at[c]], epl_hbm.at[knt_ix.at[c]],
                epl_hbm.at[epu_ix.at[c]], epl_hbm.at[epi_ix.at[c]],
                epl_hbm.at[epj_ix.at[c]],
            )
            return [pltpu.async_copy(s, rowbuf.at[r], sem_b)
                    for r, s in enumerate(srcs)]

        def compute_chunk(c, loss_acc):
            def group(g, acc):
                off = g * _L
                rows16 = off + iota
                bvec = mbb[pl.ds(c * _CHUNK + off, _L)]
                rpv = rp[pl.ds(c * _CHUNK + off, _L)]
                rnv = rn[pl.ds(c * _CHUNK + off, _L)]
                zeros = jnp.zeros((_L,), jnp.float32)

                def kstep(k, accs):
                    a1, a2, a3, a4, a5, a6, a7 = accs
                    ck = jnp.full((_L,), k, jnp.int32)

                    def row(r):
                        return plsc.load_gather(
                            rowbuf,
                            [jnp.full((_L,), r, jnp.int32), rows16, ck])

                    vhu, vti, vtj = row(0), row(1), row(2)
                    vr = plsc.load_gather(relmb_v, [bvec, ck])
                    vhp, vtp = row(3), row(4)
                    vhn, vtn = row(5), row(6)
                    vrp = plsc.load_gather(relep_v, [rpv, ck])
                    vrn = plsc.load_gather(relep_v, [rnv, ck])
                    veu, vei, vej = row(7), row(8), row(9)
                    s = vhu + vr
                    d1 = s - vti
                    d2 = s - vtj
                    e1 = vhp + vrp - vtp
                    e2 = vhn + vrn - vtn
                    vu = vhu + veu
                    f1 = vu - (vti + vei)
                    f2 = vu - (vtj + vej)
                    return (a1 + d1 * d1, a2 + d2 * d2, a3 + e1 * e1,
                            a4 + e2 * e2, a5 + f1 * f1, a6 + f2 * f2,
                            a7 + vr * vr)

                a1, a2, a3, a4, a5, a6, a7 = lax.fori_loop(
                    0, _DIM, kstep, (zeros,) * 7, unroll=2)
                pd, nd = _fsqrt(a1), _fsqrt(a2)
                pe, ne = _fsqrt(a3), _fsqrt(a4)
                dp, dn = _fsqrt(a5), _fsqrt(a6)
                rno = _fsqrt(a7)
                m = 1.5 - 1.0 / (1.0 + jnp.exp(-rno))
                t = (jnp.maximum(pd - nd + _KGE_MARGIN, 0.0)
                     + jnp.maximum(pe - ne + _KGE_MARGIN, 0.0)
                     + jnp.maximum(dp - dn + m, 0.0))
                return acc + t

            return lax.fori_loop(0, ng, group, loss_acc)

        loss_acc = jnp.zeros((_L,), jnp.float32)
        for c in range(n_chunks):
            hs = fire(c)
            for h in hs:
                h.wait()
            loss_acc = compute_chunk(c, loss_acc)

        loss_v[...] = loss_acc
        pltpu.sync_copy(loss_v, out_hbm.at[wid])

    return body


def kernel(u_batch, i_batch, j_batch, interactions, kg_pos, kg_neg,
           user_entity_map, item_entity_map,
           mbl_entities, mbl_relations, epl_entities, epl_relations):
    batch = u_batch.shape[0]
    n_chunks = batch // (_NW * _CHUNK)
    bpw = n_chunks * _CHUNK

    u32 = u_batch.astype(jnp.int32)
    i32 = i_batch.astype(jnp.int32)
    j32 = j_batch.astype(jnp.int32)
    inter_flat = interactions.astype(jnp.int32).reshape(-1)
    kgp_flat = kg_pos.astype(jnp.int32).reshape(-1)
    kgn_flat = kg_neg.astype(jnp.int32).reshape(-1)
    um1 = user_entity_map[:, 1].astype(jnp.int32)
    im1 = item_entity_map[:, 1].astype(jnp.int32)

    scratch = (
        [pltpu.VMEM((n_chunks, _CHUNK), jnp.int32)] * 12
        + [pltpu.VMEM((bpw,), jnp.int32)] * 3
        + [pltpu.VMEM((bpw * 3,), jnp.int32)] * 3
        + [pltpu.VMEM((_MBL_REL_SIZE, _DIM), jnp.float32),
           pltpu.VMEM((_REL_SIZE, _DIM), jnp.float32),
           pltpu.VMEM((10, _CHUNK, _DIM), jnp.float32),
           pltpu.VMEM((_L,), jnp.float32),
           pltpu.SemaphoreType.DMA, pltpu.SemaphoreType.DMA]
    )
    out = pl.kernel(
        _make_body(n_chunks),
        out_type=jax.ShapeDtypeStruct((_NW, _L), jnp.float32),
        mesh=plsc.VectorSubcoreMesh(core_axis_name="c", subcore_axis_name="s",
                                    num_cores=_NC, num_subcores=_NS),
        compiler_params=pltpu.CompilerParams(needs_layout_passes=False,
                                             use_tc_tiling_on_sc=False),
        scratch_types=scratch,
    )(u32, i32, j32, inter_flat, kgp_flat, kgn_flat, um1, im1,
      mbl_entities, mbl_relations, epl_entities, epl_relations)
    return jnp.sum(out) * (1.0 / batch)
